# Initial kernel scaffold; baseline (speedup 1.0000x reference)
import jax
import jax.numpy as jnp
from jax.experimental import pallas as pl


def kernel(x, embed):
    raise NotImplementedError("write your pallas kernel here")


# speedup vs baseline: 1.1870x; 1.1870x over previous
import functools

import jax
import jax.numpy as jnp
from jax import lax
from jax.experimental import pallas as pl
from jax.experimental.pallas import tpu as pltpu
from jax.experimental.pallas import tpu_sc as plsc

D = 32
NE = 8192
NTOK = 16384
TBLK = 1024
CCHUNK = 2048
EPSV = 1e-05


def _argmin_body(x_ref, e_ref, xn_ref, ind_ref):
    xs = x_ref[0]
    nrm = jnp.sqrt(jnp.sum(xs * xs, axis=1, keepdims=True))
    xn = xs / jnp.maximum(nrm, 1e-12)
    xn_ref[0] = xn
    rows_sq = jnp.sum(xn * xn, axis=1, keepdims=True)

    def chunk_max(c):
        ec = e_ref[:, pl.ds(c * CCHUNK, CCHUNK)]
        esq = jnp.sum(ec * ec, axis=0, keepdims=True)
        mm = lax.dot_general(xn.astype(jnp.bfloat16), ec.astype(jnp.bfloat16),
                             (((1,), (0,)), ((), ())),
                             preferred_element_type=jnp.float32)
        neg = -(rows_sq - 2.0 * mm + esq)
        m = jnp.max(neg, axis=1, keepdims=True)
        iota = lax.broadcasted_iota(jnp.int32, (TBLK, CCHUNK), 1)
        idx = jnp.min(jnp.where(neg == m, iota, CCHUNK), axis=1,
                      keepdims=True) + c * CCHUNK
        return m, idx

    halves = []
    for h in range(2):
        hv, hi = chunk_max(2 * h)
        for c2 in (1,):
            m, idx = chunk_max(2 * h + c2)
            upd = m > hv
            hv = jnp.where(upd, m, hv)
            hi = jnp.where(upd, idx, hi)
        halves.append((hv, hi))
    (v1, i1), (v2, i2) = halves
    v1b = v1.astype(jnp.bfloat16).astype(jnp.float32)
    take2 = v2 > v1b
    ind_ref[0] = jnp.where(take2, i2, i1)


def _run_argmin(x, embed):
    nblk = NTOK // TBLK
    return pl.pallas_call(
        _argmin_body,
        grid=(nblk,),
        in_specs=[
            pl.BlockSpec((1, TBLK, D), lambda i: (i, 0, 0)),
            pl.BlockSpec((D, NE), lambda i: (0, 0)),
        ],
        out_specs=[
            pl.BlockSpec((1, TBLK, D), lambda i: (i, 0, 0)),
            pl.BlockSpec((1, TBLK, 1), lambda i: (i, 0, 0)),
        ],
        out_shape=[
            jax.ShapeDtypeStruct((nblk, TBLK, D), jnp.float32),
            jax.ShapeDtypeStruct((nblk, TBLK, 1), jnp.int32),
        ],
    )(x.reshape(nblk, TBLK, D), embed)


def _sc_body(et_hbm, ind_hbm, q_hbm, hist_hbm, idx_v, rows_v, hist_v, sem):
    nw = 32
    bpw = NTOK // nw
    wid = lax.axis_index("s") * 2 + lax.axis_index("c")
    base = wid * bpw
    pltpu.sync_copy(ind_hbm.at[pl.ds(base, bpw)], idx_v)
    for j in range(bpw // 128):
        pltpu.async_copy(et_hbm.at[idx_v.at[pl.ds(j * 128, 128)]],
                         rows_v.at[pl.ds(j * 128, 128)], sem).wait()
    pltpu.sync_copy(rows_v, q_hbm.at[pl.ds(base, bpw)])

    def zero(i, _):
        hist_v[pl.ds(i * 16, 16)] = jnp.zeros((16,), jnp.float32)
        return 0
    lax.fori_loop(0, NE // 16, zero, 0)

    ones = jnp.full((16,), 1.0, jnp.float32)

    def acc(i, _):
        idx16 = idx_v[pl.ds(i * 16, 16)]
        plsc.addupdate_scatter(hist_v, [idx16], ones)
        return 0
    lax.fori_loop(0, bpw // 16, acc, 0)
    pltpu.sync_copy(hist_v, hist_hbm.at[wid])


def _run_sc(embed_t, ind_flat):
    nw = 32
    bpw = NTOK // nw
    mesh = plsc.VectorSubcoreMesh(core_axis_name="c", subcore_axis_name="s")
    f = functools.partial(
        pl.kernel, mesh=mesh,
        compiler_params=pltpu.CompilerParams(needs_layout_passes=False,
                                             use_tc_tiling_on_sc=False),
        out_type=[
            jax.ShapeDtypeStruct((NTOK, D), jnp.float32),
            jax.ShapeDtypeStruct((nw, NE), jnp.float32),
        ],
        scratch_types=[
            pltpu.VMEM((bpw,), jnp.int32),
            pltpu.VMEM((bpw, D), jnp.float32),
            pltpu.VMEM((NE,), jnp.float32),
            pltpu.SemaphoreType.DMA,
        ],
    )(_sc_body)
    return f(embed_t, ind_flat)


def _epi_body(xn_ref, q_ref, h_ref, qst_ref, diff_ref, prob_ref, perp_ref):
    xn = xn_ref[...]
    q = q_ref[...]
    d = q - xn
    qst_ref[...] = xn + d
    diff_ref[...] = jnp.reshape(jnp.mean(jnp.abs(d)), (1, 1))
    counts = jnp.sum(h_ref[...], axis=0, keepdims=True)
    prob = counts / (jnp.sum(counts) + EPSV)
    prob_ref[...] = prob
    plogp = prob * jnp.log(jnp.maximum(prob, EPSV))
    perp_ref[...] = jnp.reshape(jnp.exp(-jnp.sum(plogp)), (1, 1))


def _run_epi(xn_flat, q_flat, part_hist):
    return pl.pallas_call(
        _epi_body,
        out_shape=[
            jax.ShapeDtypeStruct((NTOK, D), jnp.float32),
            jax.ShapeDtypeStruct((1, 1), jnp.float32),
            jax.ShapeDtypeStruct((1, NE), jnp.float32),
            jax.ShapeDtypeStruct((1, 1), jnp.float32),
        ],
    )(xn_flat, q_flat, part_hist)


def kernel(x, embed):
    b, t, d = x.shape
    xn3, ind3 = _run_argmin(x, embed)
    ind_flat = ind3.reshape(NTOK)
    quant, part_hist = _run_sc(embed.T, ind_flat)
    qst, diff, prob, perp = _run_epi(xn3.reshape(NTOK, D), quant, part_hist)
    return (qst.reshape(b, t, d), diff[0, 0], ind3.reshape(b, t),
            prob.reshape(NE), perp[0, 0])


# speedup vs baseline: 1.3494x; 1.1368x over previous
import functools

import jax
import jax.numpy as jnp
from jax import lax
from jax.experimental import pallas as pl
from jax.experimental.pallas import tpu as pltpu
from jax.experimental.pallas import tpu_sc as plsc

D = 32
NE = 8192
NTOK = 16384
TBLK = 1024
CCHUNK = 2048
EPSV = 1e-05


def _argmin_body(x_ref, e_ref, xn_ref, ind_ref):
    xs = x_ref[0]
    nrm = jnp.sqrt(jnp.sum(xs * xs, axis=1, keepdims=True))
    xn = xs / jnp.maximum(nrm, 1e-12)
    xn_ref[0] = xn
    rows_sq = jnp.sum(xn * xn, axis=1, keepdims=True)

    iota = lax.broadcasted_iota(jnp.int32, (TBLK, CCHUNK), 1)

    def chunk_min(c):
        ec = e_ref[:, pl.ds(c * CCHUNK, CCHUNK)]
        esq = jnp.sum(ec * ec, axis=0, keepdims=True)
        mm = lax.dot_general(xn.astype(jnp.bfloat16), ec.astype(jnp.bfloat16),
                             (((1,), (0,)), ((), ())),
                             preferred_element_type=jnp.float32)
        dist = (rows_sq - 2.0 * mm) + esq
        m = jnp.min(dist, axis=1, keepdims=True)
        idx = jnp.min(jnp.where(dist == m, iota, CCHUNK), axis=1,
                      keepdims=True) + c * CCHUNK
        return m, idx

    halves = []
    for h in range(2):
        hv, hi = chunk_min(2 * h)
        m, idx = chunk_min(2 * h + 1)
        upd = m < hv
        hv = jnp.where(upd, m, hv)
        hi = jnp.where(upd, idx, hi)
        halves.append((hv, hi))
    (v1, i1), (v2, i2) = halves
    v1b = v1.astype(jnp.bfloat16).astype(jnp.float32)
    take2 = v2 < v1b
    ind_ref[0] = jnp.where(take2, i2, i1)


def _run_argmin(x, embed):
    nblk = NTOK // TBLK
    return pl.pallas_call(
        _argmin_body,
        grid=(nblk,),
        in_specs=[
            pl.BlockSpec((1, TBLK, D), lambda i: (i, 0, 0)),
            pl.BlockSpec((D, NE), lambda i: (0, 0)),
        ],
        out_specs=[
            pl.BlockSpec((1, TBLK, D), lambda i: (i, 0, 0)),
            pl.BlockSpec((1, TBLK, 1), lambda i: (i, 0, 0)),
        ],
        out_shape=[
            jax.ShapeDtypeStruct((nblk, TBLK, D), jnp.float32),
            jax.ShapeDtypeStruct((nblk, TBLK, 1), jnp.int32),
        ],
    )(x.reshape(nblk, TBLK, D), embed)


def _sc_body(et_hbm, ind_hbm, q_hbm, hist_hbm, idx_v, rows_v, hist_v, sem):
    nw = 32
    bpw = NTOK // nw
    wid = lax.axis_index("s") * 2 + lax.axis_index("c")
    base = wid * bpw
    pltpu.sync_copy(ind_hbm.at[pl.ds(base, bpw)], idx_v)
    for j in range(bpw // 128):
        pltpu.async_copy(et_hbm.at[idx_v.at[pl.ds(j * 128, 128)]],
                         rows_v.at[pl.ds(j * 128, 128)], sem).wait()
    pltpu.sync_copy(rows_v, q_hbm.at[pl.ds(base, bpw)])

    def zero(i, _):
        hist_v[pl.ds(i * 16, 16)] = jnp.zeros((16,), jnp.float32)
        return 0
    lax.fori_loop(0, NE // 16, zero, 0)

    ones = jnp.full((16,), 1.0, jnp.float32)

    def acc(i, _):
        idx16 = idx_v[pl.ds(i * 16, 16)]
        plsc.addupdate_scatter(hist_v, [idx16], ones)
        return 0
    lax.fori_loop(0, bpw // 16, acc, 0)
    pltpu.sync_copy(hist_v, hist_hbm.at[wid])


def _run_sc(embed_t, ind_flat):
    nw = 32
    bpw = NTOK // nw
    mesh = plsc.VectorSubcoreMesh(core_axis_name="c", subcore_axis_name="s")
    f = functools.partial(
        pl.kernel, mesh=mesh,
        compiler_params=pltpu.CompilerParams(needs_layout_passes=False,
                                             use_tc_tiling_on_sc=False),
        out_type=[
            jax.ShapeDtypeStruct((NTOK, D), jnp.float32),
            jax.ShapeDtypeStruct((nw, NE), jnp.float32),
        ],
        scratch_types=[
            pltpu.VMEM((bpw,), jnp.int32),
            pltpu.VMEM((bpw, D), jnp.float32),
            pltpu.VMEM((NE,), jnp.float32),
            pltpu.SemaphoreType.DMA,
        ],
    )(_sc_body)
    return f(embed_t, ind_flat)


def _epi_body(xn_ref, q_ref, h_ref, qst_ref, diff_ref, prob_ref, perp_ref):
    xn = xn_ref[...]
    q = q_ref[...]
    d = q - xn
    qst_ref[...] = xn + d
    diff_ref[...] = jnp.reshape(jnp.mean(jnp.abs(d)), (1, 1))
    counts = jnp.sum(h_ref[...], axis=0, keepdims=True)
    prob = counts / (jnp.sum(counts) + EPSV)
    prob_ref[...] = prob
    plogp = prob * jnp.log(jnp.maximum(prob, EPSV))
    perp_ref[...] = jnp.reshape(jnp.exp(-jnp.sum(plogp)), (1, 1))


def _run_epi(xn_flat, q_flat, part_hist):
    return pl.pallas_call(
        _epi_body,
        out_shape=[
            jax.ShapeDtypeStruct((NTOK, D), jnp.float32),
            jax.ShapeDtypeStruct((1, 1), jnp.float32),
            jax.ShapeDtypeStruct((1, NE), jnp.float32),
            jax.ShapeDtypeStruct((1, 1), jnp.float32),
        ],
    )(xn_flat, q_flat, part_hist)


def kernel(x, embed):
    b, t, d = x.shape
    xn3, ind3 = _run_argmin(x, embed)
    ind_flat = ind3.reshape(NTOK)
    quant, part_hist = _run_sc(embed.T, ind_flat)
    qst, diff, prob, perp = _run_epi(xn3.reshape(NTOK, D), quant, part_hist)
    return (qst.reshape(b, t, d), diff[0, 0], ind3.reshape(b, t),
            prob.reshape(NE), perp[0, 0])
